# baseline (speedup 1.0000x reference)
import functools

import jax
import jax.numpy as jnp
from jax import lax
from jax.experimental import pallas as pl
from jax.experimental.pallas import tpu as pltpu
from jax.experimental.pallas import tpu_sc as plsc

_T = 8192
_D = 768
_F = 3072
_E = 8

_TM = 256
_NTILES = _T // _TM
_NWORK = _NTILES + _E - 1

_NC = 2
_NS = 16
_NW = _NC * _NS
_RPW = _T // _NW
_CH = 64
_NCH = _RPW // _CH


def _router_body(x_ref, wr_ref, idx_ref):
    l = jnp.dot(x_ref[...], wr_ref[...], preferred_element_type=jnp.float32)
    idx_ref[0, 0, :] = jnp.argmax(l, axis=-1).astype(jnp.int32)


_RB = 1024


def _router(x, Wr):
    return pl.pallas_call(
        _router_body,
        grid=(_T // _RB,),
        in_specs=[
            pl.BlockSpec((_RB, _D), lambda i: (i, 0)),
            pl.BlockSpec((_D, _E), lambda i: (0, 0)),
        ],
        out_specs=pl.BlockSpec((1, 1, _RB), lambda i: (i, 0, 0)),
        out_shape=jax.ShapeDtypeStruct((_T // _RB, 1, _RB), jnp.int32),
    )(x, Wr)


def _metadata(goff):
    start, end = goff[:-1], goff[1:]
    nonempty = end > start
    first_t = start // _TM
    last_t = jnp.where(nonempty, (end - 1) // _TM, first_t)
    items = jnp.where(nonempty, last_t - first_t + 1, 0)
    ib = jnp.concatenate(
        [jnp.zeros((1,), jnp.int32), jnp.cumsum(items).astype(jnp.int32)])
    j = jnp.arange(_NWORK, dtype=jnp.int32)
    total = ib[_E]
    gsel = jnp.clip(
        jnp.searchsorted(ib, j, side="right").astype(jnp.int32) - 1, 0, _E - 1)
    tile = first_t[gsel] + (j - ib[gsel])
    valid = (j < total).astype(jnp.int32)
    tile = jnp.where(valid == 1, tile, _NTILES - 1).astype(jnp.int32)
    prev = jnp.concatenate([jnp.full((1,), -1, jnp.int32), tile[:-1]])
    ini = ((valid == 1) & (tile != prev)).astype(jnp.int32)
    return gsel.astype(jnp.int32), tile, valid, ini, goff


def _ffn_body(gid_ref, tid_ref, vld_ref, ini_ref, goff_ref,
              xs_ref, wr_ref, w1_ref, w2_ref, out_ref):
    i = pl.program_id(0)
    g = gid_ref[i]
    rows = tid_ref[i] * _TM + lax.broadcasted_iota(jnp.int32, (_TM, 1), 0)
    m = (rows >= goff_ref[g]) & (rows < goff_ref[g + 1]) & (vld_ref[i] > 0)
    l = jnp.dot(xs_ref[...], wr_ref[...], preferred_element_type=jnp.float32)
    mx = jnp.max(l, axis=-1)
    gate = 1.0 / jnp.sum(jnp.exp(l - mx[:, None]), axis=-1)
    xg = jnp.where(m, xs_ref[...] * gate[:, None], 0.0)
    h = jnp.maximum(
        jnp.dot(xg, w1_ref[0, :, :], preferred_element_type=jnp.float32), 0.0)
    c = jnp.dot(h, w2_ref[0, :, :], preferred_element_type=jnp.float32)

    @pl.when(ini_ref[i] > 0)
    def _():
        out_ref[...] = xs_ref[...] + c

    @pl.when(ini_ref[i] == 0)
    def _():
        out_ref[...] = out_ref[...] + c


def _ffn(xs, Wr, W1, W2, gid, tid, vld, ini, goff):
    grid_spec = pltpu.PrefetchScalarGridSpec(
        num_scalar_prefetch=5,
        grid=(_NWORK,),
        in_specs=[
            pl.BlockSpec((_TM, _D),
                         lambda i, gid, tid, vld, ini, goff: (tid[i], 0)),
            pl.BlockSpec((_D, _E),
                         lambda i, gid, tid, vld, ini, goff: (0, 0)),
            pl.BlockSpec((1, _D, _F),
                         lambda i, gid, tid, vld, ini, goff: (gid[i], 0, 0)),
            pl.BlockSpec((1, _F, _D),
                         lambda i, gid, tid, vld, ini, goff: (gid[i], 0, 0)),
        ],
        out_specs=pl.BlockSpec(
            (_TM, _D), lambda i, gid, tid, vld, ini, goff: (tid[i], 0)),
    )
    return pl.pallas_call(
        _ffn_body,
        grid_spec=grid_spec,
        out_shape=jax.ShapeDtypeStruct((_T, _D), jnp.float32),
        compiler_params=pltpu.CompilerParams(
            dimension_semantics=("arbitrary",)),
    )(gid, tid, vld, ini, goff, xs, Wr, W1, W2)


def _gather_body(x_hbm, perm_hbm, xs_hbm, idx_v, rows_v, sem):
    wid = lax.axis_index("s") * _NC + lax.axis_index("c")
    base = wid * _RPW
    for c in range(_NCH):
        cb = base + c * _CH
        pltpu.sync_copy(perm_hbm.at[pl.ds(cb, _CH)], idx_v)
        pltpu.async_copy(x_hbm.at[idx_v], rows_v, sem).wait()
        pltpu.sync_copy(rows_v, xs_hbm.at[pl.ds(cb, _CH)])


@functools.cache
def _gather():
    mesh = plsc.VectorSubcoreMesh(core_axis_name="c", subcore_axis_name="s")
    return pl.kernel(
        _gather_body,
        out_type=jax.ShapeDtypeStruct((_T, _D), jnp.float32),
        mesh=mesh,
        scratch_types=[
            pltpu.VMEM((_CH,), jnp.int32),
            pltpu.VMEM((_CH, _D), jnp.float32),
            pltpu.SemaphoreType.DMA,
        ],
    )


def _scatter_body(ys_hbm, perm_hbm, out_hbm, idx_v, rows_v, sem):
    wid = lax.axis_index("s") * _NC + lax.axis_index("c")
    base = wid * _RPW
    for c in range(_NCH):
        cb = base + c * _CH
        pltpu.sync_copy(perm_hbm.at[pl.ds(cb, _CH)], idx_v)
        pltpu.sync_copy(ys_hbm.at[pl.ds(cb, _CH)], rows_v)
        pltpu.async_copy(rows_v, out_hbm.at[idx_v], sem).wait()


@functools.cache
def _scatter():
    mesh = plsc.VectorSubcoreMesh(core_axis_name="c", subcore_axis_name="s")
    return pl.kernel(
        _scatter_body,
        out_type=jax.ShapeDtypeStruct((_T, _D), jnp.float32),
        mesh=mesh,
        scratch_types=[
            pltpu.VMEM((_CH,), jnp.int32),
            pltpu.VMEM((_CH, _D), jnp.float32),
            pltpu.SemaphoreType.DMA,
        ],
    )


def kernel(x, Wr, W1, W2):
    idx3 = _router(x, Wr)
    idx = idx3.reshape(_T)
    idx_s, perm = lax.sort_key_val(idx, jnp.arange(_T, dtype=jnp.int32))
    goff = jnp.searchsorted(
        idx_s, jnp.arange(_E + 1, dtype=jnp.int32), side="left"
    ).astype(jnp.int32)
    gid, tid, vld, ini, goff = _metadata(goff)
    xs = _gather()(x, perm)
    ys = _ffn(xs, Wr, W1, W2, gid, tid, vld, ini, goff)
    return _scatter()(ys, perm)


# speedup vs baseline: 1.0409x; 1.0409x over previous
import functools

import jax
import jax.numpy as jnp
from jax import lax
from jax.experimental import pallas as pl
from jax.experimental.pallas import tpu as pltpu
from jax.experimental.pallas import tpu_sc as plsc

_T = 8192
_D = 768
_F = 3072
_E = 8

_TM = 512
_NTILES = _T // _TM
_NWORK = _NTILES + _E - 1

_NC = 2
_NS = 16
_NW = _NC * _NS
_RPW = _T // _NW
_CH = 64
_NCH = _RPW // _CH


def _router_body(x_ref, wr_ref, idx_ref):
    l = jnp.dot(x_ref[...], wr_ref[...], preferred_element_type=jnp.float32)
    idx_ref[0, 0, :] = jnp.argmax(l, axis=-1).astype(jnp.int32)


_RB = 1024


def _router(x, Wr):
    return pl.pallas_call(
        _router_body,
        grid=(_T // _RB,),
        in_specs=[
            pl.BlockSpec((_RB, _D), lambda i: (i, 0)),
            pl.BlockSpec((_D, _E), lambda i: (0, 0)),
        ],
        out_specs=pl.BlockSpec((1, 1, _RB), lambda i: (i, 0, 0)),
        out_shape=jax.ShapeDtypeStruct((_T // _RB, 1, _RB), jnp.int32),
    )(x, Wr)


def _metadata(goff):
    start, end = goff[:-1], goff[1:]
    nonempty = end > start
    first_t = start // _TM
    last_t = jnp.where(nonempty, (end - 1) // _TM, first_t)
    items = jnp.where(nonempty, last_t - first_t + 1, 0)
    ib = jnp.concatenate(
        [jnp.zeros((1,), jnp.int32), jnp.cumsum(items).astype(jnp.int32)])
    j = jnp.arange(_NWORK, dtype=jnp.int32)
    total = ib[_E]
    gsel = jnp.clip(
        jnp.searchsorted(ib, j, side="right").astype(jnp.int32) - 1, 0, _E - 1)
    tile = first_t[gsel] + (j - ib[gsel])
    valid = (j < total).astype(jnp.int32)
    tile = jnp.where(valid == 1, tile, _NTILES - 1).astype(jnp.int32)
    prev = jnp.concatenate([jnp.full((1,), -1, jnp.int32), tile[:-1]])
    ini = ((valid == 1) & (tile != prev)).astype(jnp.int32)
    return gsel.astype(jnp.int32), tile, valid, ini, goff


def _ffn_body(gid_ref, tid_ref, vld_ref, ini_ref, goff_ref,
              xs_ref, wr_ref, w1_ref, w2_ref, out_ref):
    i = pl.program_id(0)
    g = gid_ref[i]
    rows = tid_ref[i] * _TM + lax.broadcasted_iota(jnp.int32, (_TM, 1), 0)
    m = (rows >= goff_ref[g]) & (rows < goff_ref[g + 1]) & (vld_ref[i] > 0)
    l = jnp.dot(xs_ref[...], wr_ref[...], preferred_element_type=jnp.float32)
    mx = jnp.max(l, axis=-1)
    gate = 1.0 / jnp.sum(jnp.exp(l - mx[:, None]), axis=-1)
    xg = jnp.where(m, xs_ref[...] * gate[:, None], 0.0)
    h = jnp.maximum(
        jnp.dot(xg, w1_ref[0, :, :], preferred_element_type=jnp.float32), 0.0)
    c = jnp.dot(h, w2_ref[0, :, :], preferred_element_type=jnp.float32)

    @pl.when(ini_ref[i] > 0)
    def _():
        out_ref[...] = xs_ref[...] + c

    @pl.when(ini_ref[i] == 0)
    def _():
        out_ref[...] = out_ref[...] + c


def _ffn(xs, Wr, W1, W2, gid, tid, vld, ini, goff):
    grid_spec = pltpu.PrefetchScalarGridSpec(
        num_scalar_prefetch=5,
        grid=(_NWORK,),
        in_specs=[
            pl.BlockSpec((_TM, _D),
                         lambda i, gid, tid, vld, ini, goff: (tid[i], 0)),
            pl.BlockSpec((_D, _E),
                         lambda i, gid, tid, vld, ini, goff: (0, 0)),
            pl.BlockSpec((1, _D, _F),
                         lambda i, gid, tid, vld, ini, goff: (gid[i], 0, 0)),
            pl.BlockSpec((1, _F, _D),
                         lambda i, gid, tid, vld, ini, goff: (gid[i], 0, 0)),
        ],
        out_specs=pl.BlockSpec(
            (_TM, _D), lambda i, gid, tid, vld, ini, goff: (tid[i], 0)),
    )
    return pl.pallas_call(
        _ffn_body,
        grid_spec=grid_spec,
        out_shape=jax.ShapeDtypeStruct((_T, _D), jnp.float32),
        compiler_params=pltpu.CompilerParams(
            dimension_semantics=("arbitrary",)),
    )(gid, tid, vld, ini, goff, xs, Wr, W1, W2)


def _gather_body(x_hbm, perm_hbm, xs_hbm, idx_v, rows_v, sem):
    wid = lax.axis_index("s") * _NC + lax.axis_index("c")
    base = wid * _RPW
    for c in range(_NCH):
        cb = base + c * _CH
        pltpu.sync_copy(perm_hbm.at[pl.ds(cb, _CH)], idx_v)
        pltpu.async_copy(x_hbm.at[idx_v], rows_v, sem).wait()
        pltpu.sync_copy(rows_v, xs_hbm.at[pl.ds(cb, _CH)])


@functools.cache
def _gather():
    mesh = plsc.VectorSubcoreMesh(core_axis_name="c", subcore_axis_name="s")
    return pl.kernel(
        _gather_body,
        out_type=jax.ShapeDtypeStruct((_T, _D), jnp.float32),
        mesh=mesh,
        scratch_types=[
            pltpu.VMEM((_CH,), jnp.int32),
            pltpu.VMEM((_CH, _D), jnp.float32),
            pltpu.SemaphoreType.DMA,
        ],
    )


def _scatter_body(ys_hbm, perm_hbm, out_hbm, idx_v, rows_v, sem):
    wid = lax.axis_index("s") * _NC + lax.axis_index("c")
    base = wid * _RPW
    for c in range(_NCH):
        cb = base + c * _CH
        pltpu.sync_copy(perm_hbm.at[pl.ds(cb, _CH)], idx_v)
        pltpu.sync_copy(ys_hbm.at[pl.ds(cb, _CH)], rows_v)
        pltpu.async_copy(rows_v, out_hbm.at[idx_v], sem).wait()


@functools.cache
def _scatter():
    mesh = plsc.VectorSubcoreMesh(core_axis_name="c", subcore_axis_name="s")
    return pl.kernel(
        _scatter_body,
        out_type=jax.ShapeDtypeStruct((_T, _D), jnp.float32),
        mesh=mesh,
        scratch_types=[
            pltpu.VMEM((_CH,), jnp.int32),
            pltpu.VMEM((_CH, _D), jnp.float32),
            pltpu.SemaphoreType.DMA,
        ],
    )


def kernel(x, Wr, W1, W2):
    idx3 = _router(x, Wr)
    idx = idx3.reshape(_T)
    idx_s, perm = lax.sort_key_val(idx, jnp.arange(_T, dtype=jnp.int32))
    goff = jnp.searchsorted(
        idx_s, jnp.arange(_E + 1, dtype=jnp.int32), side="left"
    ).astype(jnp.int32)
    gid, tid, vld, ini, goff = _metadata(goff)
    xs = _gather()(x, perm)
    ys = _ffn(xs, Wr, W1, W2, gid, tid, vld, ini, goff)
    return _scatter()(ys, perm)
